# baseline (speedup 1.0000x reference)
import functools

import jax
import jax.numpy as jnp
from jax import lax
from jax.experimental import pallas as pl
from jax.experimental.pallas import tpu as pltpu
from jax.experimental.pallas import tpu_sc as plsc


def _transform_body(tbl_ref, w_ref, b_ref, out_ref):
    x = tbl_ref[...]
    y = lax.dot_general(
        x, w_ref[...], (((1,), (1,)), ((), ())),
        preferred_element_type=jnp.float32,
    )
    y = y + b_ref[...]
    out_ref[...] = y * jax.nn.sigmoid(y)


def _transform_table(emb_table, W, b):
    V, D = emb_table.shape
    BLK = 200 if V % 200 == 0 else V
    n_blk = V // BLK
    out = pl.pallas_call(
        _transform_body,
        grid=(n_blk,),
        in_specs=[
            pl.BlockSpec((BLK, D), lambda i: (i, 0)),
            pl.BlockSpec((D, D), lambda i: (0, 0)),
            pl.BlockSpec((1, D), lambda i: (0, 0)),
        ],
        out_specs=pl.BlockSpec((BLK, D), lambda i: (i, 0)),
        out_shape=jax.ShapeDtypeStruct((V, D), jnp.float32),
    )(emb_table, W, b.reshape(1, D))
    return out


@functools.lru_cache(maxsize=None)
def _make_gather(V, D, B):
    info = plsc.get_sparse_core_info()
    NC, NS = info.num_cores, info.num_subcores
    NW = NC * NS
    b_per_w = B // NW
    CHUNK = 64
    n_chunks = b_per_w // CHUNK
    STG = -(-V // (8 * NS)) * 8

    mesh = plsc.VectorSubcoreMesh(core_axis_name="c", subcore_axis_name="s")

    @functools.partial(
        pl.kernel,
        mesh=mesh,
        out_type=jax.ShapeDtypeStruct((B, D), jnp.float32),
        scratch_types=[
            pltpu.VMEM((b_per_w,), jnp.int32),
            pltpu.VMEM((b_per_w, D), jnp.float32),
            pltpu.VMEM_SHARED((V, D), jnp.float32),
        ]
        + [pltpu.SemaphoreType.DMA] * n_chunks
        + [pltpu.SemaphoreType.DMA],
    )
    def gather(tbl_hbm, idx_hbm, out_hbm, idx_v, rows_v, tbl_sp, *sems):
        g_sems, w_sem = sems[:n_chunks], sems[n_chunks]
        sid = lax.axis_index("s")
        wid = sid * NC + lax.axis_index("c")
        base = wid * b_per_w
        pltpu.sync_copy(idx_hbm.at[pl.ds(base, b_per_w)], idx_v)
        gathers = [
            pltpu.async_copy(
                tbl_hbm.at[idx_v.at[pl.ds(0, CHUNK)]],
                rows_v.at[pl.ds(0, CHUNK)],
                g_sems[0],
            )
        ]
        r0 = lax.min(sid * STG, jnp.int32(V - STG))
        pltpu.sync_copy(tbl_hbm.at[pl.ds(r0, STG)], tbl_sp.at[pl.ds(r0, STG)])

        plsc.subcore_barrier()
        gathers += [
            pltpu.async_copy(
                tbl_sp.at[idx_v.at[pl.ds(j * CHUNK, CHUNK)]],
                rows_v.at[pl.ds(j * CHUNK, CHUNK)],
                g_sems[j],
            )
            for j in range(1, n_chunks)
        ]
        writes = []
        for j in range(n_chunks):
            gathers[j].wait()
            writes.append(
                pltpu.async_copy(
                    rows_v.at[pl.ds(j * CHUNK, CHUNK)],
                    out_hbm.at[pl.ds(base + j * CHUNK, CHUNK)],
                    w_sem,
                )
            )
        for c in writes:
            c.wait()

    return gather


def kernel(t, emb_table, W, b):
    V, D = emb_table.shape
    B = t.shape[0]
    tbl2 = _transform_table(emb_table, W, b)
    return _make_gather(V, D, B)(tbl2, t.astype(jnp.int32))


# speedup vs baseline: 1.0668x; 1.0668x over previous
import functools

import jax
import jax.numpy as jnp
from jax import lax
from jax.experimental import pallas as pl
from jax.experimental.pallas import tpu as pltpu
from jax.experimental.pallas import tpu_sc as plsc


def _transform_body(tbl_ref, w_ref, b_ref, out_ref):
    x = tbl_ref[...]
    y = lax.dot_general(
        x, w_ref[...], (((1,), (1,)), ((), ())),
        preferred_element_type=jnp.float32,
    )
    y = y + b_ref[...]
    out_ref[...] = y * jax.nn.sigmoid(y)


def _transform_table(emb_table, W, b):
    V, D = emb_table.shape
    return pl.pallas_call(
        _transform_body,
        out_shape=jax.ShapeDtypeStruct((V, D), jnp.float32),
    )(emb_table, W, b.reshape(1, D))


@functools.lru_cache(maxsize=None)
def _make_gather(V, D, B):
    info = plsc.get_sparse_core_info()
    NC, NS = info.num_cores, info.num_subcores
    NW = NC * NS
    b_per_w = B // NW
    CHUNK = 64
    n_chunks = b_per_w // CHUNK
    STG = -(-V // (8 * NS)) * 8

    mesh = plsc.VectorSubcoreMesh(core_axis_name="c", subcore_axis_name="s")

    @functools.partial(
        pl.kernel,
        mesh=mesh,
        out_type=jax.ShapeDtypeStruct((B, D), jnp.float32),
        scratch_types=[
            pltpu.VMEM((b_per_w,), jnp.int32),
            pltpu.VMEM((b_per_w, D), jnp.float32),
            pltpu.VMEM_SHARED((V, D), jnp.float32),
        ]
        + [pltpu.SemaphoreType.DMA] * n_chunks
        + [pltpu.SemaphoreType.DMA],
    )
    def gather(tbl_hbm, idx_hbm, out_hbm, idx_v, rows_v, tbl_sp, *sems):
        g_sems, w_sem = sems[:n_chunks], sems[n_chunks]
        sid = lax.axis_index("s")
        wid = sid * NC + lax.axis_index("c")
        base = wid * b_per_w
        pltpu.sync_copy(idx_hbm.at[pl.ds(base, b_per_w)], idx_v)
        gathers = [
            pltpu.async_copy(
                tbl_hbm.at[idx_v.at[pl.ds(0, CHUNK)]],
                rows_v.at[pl.ds(0, CHUNK)],
                g_sems[0],
            )
        ]
        r0 = lax.min(sid * STG, jnp.int32(V - STG))
        pltpu.sync_copy(tbl_hbm.at[pl.ds(r0, STG)], tbl_sp.at[pl.ds(r0, STG)])

        plsc.subcore_barrier()
        gathers += [
            pltpu.async_copy(
                tbl_sp.at[idx_v.at[pl.ds(j * CHUNK, CHUNK)]],
                rows_v.at[pl.ds(j * CHUNK, CHUNK)],
                g_sems[j],
            )
            for j in range(1, n_chunks)
        ]
        writes = []
        for j in range(n_chunks):
            gathers[j].wait()
            writes.append(
                pltpu.async_copy(
                    rows_v.at[pl.ds(j * CHUNK, CHUNK)],
                    out_hbm.at[pl.ds(base + j * CHUNK, CHUNK)],
                    w_sem,
                )
            )
        for c in writes:
            c.wait()

    return gather


def kernel(t, emb_table, W, b):
    V, D = emb_table.shape
    B = t.shape[0]
    tbl2 = _transform_table(emb_table, W, b)
    return _make_gather(V, D, B)(tbl2, t.astype(jnp.int32))
